# Initial kernel scaffold; baseline (speedup 1.0000x reference)
import jax
import jax.numpy as jnp
from jax.experimental import pallas as pl


def kernel(conv1_w, conv1_b, conv2_w, conv2_b, conv3_w, conv3_b, fc1_w, fc1_b, fc2_w, fc2_b, x_nchw):
    raise NotImplementedError("write your pallas kernel here")


# speedup vs baseline: 1.8533x; 1.8533x over previous
import functools

import jax
import jax.numpy as jnp
from jax.experimental import pallas as pl
from jax.experimental.pallas import tpu as pltpu


def _cnn_kernel(xc_ref, w1_ref, b1_ref, w2_ref, b2_ref, w3_ref, b3_ref,
                wf1_ref, bf1_ref, wf2_ref, bf2_ref,
                o_ref,
                pad2_ref, pad3_ref, *, b_blk):
    f32 = jnp.float32
    bf16 = jnp.bfloat16
    b = b_blk

    xc = xc_ref[...].reshape(b * 32 * 8, 64)
    a1 = jnp.dot(xc, w1_ref[...], preferred_element_type=f32) + b1_ref[...]
    a1 = jnp.maximum(a1, 0.0).reshape(b, 16, 2, 8, 256)
    a1 = jnp.max(a1, axis=2)
    p1e = jnp.maximum(a1[..., 0:64], a1[..., 64:128]).astype(bf16)
    p1o = jnp.maximum(a1[..., 128:192], a1[..., 192:256]).astype(bf16)

    z_row = jnp.zeros((b, 1, 9, 128), bf16)
    pad2_ref[:, 0:1] = z_row
    pad2_ref[:, 17:18] = z_row
    pad2_ref[:, 1:17, 0:1, 0:64] = jnp.zeros((b, 16, 1, 64), bf16)
    pad2_ref[:, 1:17, 8:9, 64:128] = jnp.zeros((b, 16, 1, 64), bf16)
    pad2_ref[:, 1:17, 0:8, 64:128] = p1e
    pad2_ref[:, 1:17, 1:9, 0:64] = p1o

    x2 = jnp.concatenate(
        [pad2_ref[:, dy:dy + 16, dg:dg + 8, :].reshape(b * 16 * 8, 128)
         for dy in range(3) for dg in range(2)], axis=-1)
    a2 = jnp.dot(x2, w2_ref[...], preferred_element_type=f32) + b2_ref[...]
    a2 = jnp.maximum(a2, 0.0).reshape(b, 8, 2, 8, 256)
    a2 = jnp.max(a2, axis=2)
    p2 = jnp.maximum(a2[..., 0:128], a2[..., 128:256]).astype(bf16)

    z3_row = jnp.zeros((b, 1, 10, 128), bf16)
    pad3_ref[:, 0:1] = z3_row
    pad3_ref[:, 9:10] = z3_row
    z3_col = jnp.zeros((b, 8, 1, 128), bf16)
    pad3_ref[:, 1:9, 0:1, :] = z3_col
    pad3_ref[:, 1:9, 9:10, :] = z3_col
    pad3_ref[:, 1:9, 1:9, :] = p2
    x3 = jnp.concatenate(
        [pad3_ref[:, dy:dy + 8, dx:dx + 8, :].reshape(b * 64, 128)
         for dy in range(3) for dx in range(3)], axis=-1)
    a3 = jnp.dot(x3, w3_ref[...], preferred_element_type=f32) + b3_ref[...]
    a3 = jnp.maximum(a3, 0.0).reshape(b * 4, 2, 4, 2, 256)
    a3 = jnp.max(a3, axis=3)
    a3 = jnp.max(a3, axis=1)
    p3 = a3.reshape(b, 4, 4, 256).astype(bf16)

    feat = jnp.concatenate(
        [p3[:, hh, ww, :] for hh in range(4) for ww in range(4)], axis=-1)
    h1 = jnp.dot(feat, wf1_ref[...], preferred_element_type=f32) + bf1_ref[...]
    h1 = jnp.maximum(h1, 0.0).astype(bf16)

    logits = jnp.dot(h1, wf2_ref[...], preferred_element_type=f32) + bf2_ref[...]
    m = jnp.max(logits, axis=-1, keepdims=True)
    lse = m + jnp.log(jnp.sum(jnp.exp(logits - m), axis=-1, keepdims=True))
    o_ref[...] = logits - lse


def _prep_weights(conv1_w, conv1_b, conv2_w, conv2_b, conv3_w, conv3_b,
                  fc1_w, fc1_b, fc2_w, fc2_b):
    bf16, f32 = jnp.bfloat16, jnp.float32

    t1 = jnp.transpose(conv1_w, (2, 3, 1, 0))
    w1 = jnp.concatenate(
        [jnp.pad(t1, ((0, 0), (j, 3 - j), (0, 0), (0, 0))).reshape(54, 64)
         for j in range(4)], axis=1)
    w1 = jnp.pad(w1, ((0, 10), (0, 0))).astype(bf16)
    b1 = jnp.tile(conv1_b, 4).reshape(1, 256).astype(f32)

    t2 = jnp.transpose(conv2_w, (2, 3, 1, 0))
    w2 = jnp.concatenate(
        [jnp.pad(t2, ((0, 0), (j, 1 - j), (0, 0), (0, 0))).reshape(768, 128)
         for j in range(2)], axis=1).astype(bf16)
    b2 = jnp.tile(conv2_b, 2).reshape(1, 256).astype(f32)

    w3 = jnp.transpose(conv3_w, (2, 3, 1, 0)).reshape(1152, 256).astype(bf16)
    b3 = conv3_b.reshape(1, 256).astype(f32)

    wf1 = fc1_w.T.reshape(256, 4, 4, 512)
    wf1 = jnp.transpose(wf1, (1, 2, 0, 3)).reshape(4096, 512).astype(bf16)
    bf1 = fc1_b.reshape(1, 512).astype(f32)

    wf2 = jnp.pad(fc2_w.T, ((0, 0), (0, 118))).astype(bf16)
    bf2 = jnp.pad(fc2_b, (0, 118), constant_values=-1e9)
    bf2 = bf2.reshape(1, 128).astype(f32)

    return (w1, b1, w2, b2, w3, b3, wf1, bf1, wf2, bf2)


def kernel(conv1_w, conv1_b, conv2_w, conv2_b, conv3_w, conv3_b,
           fc1_w, fc1_b, fc2_w, fc2_b, x_nchw, *, block_b=8):
    w = _prep_weights(conv1_w, conv1_b, conv2_w, conv2_b, conv3_w, conv3_b,
                      fc1_w, fc1_b, fc2_w, fc2_b)
    B = x_nchw.shape[0]

    x = jnp.transpose(x_nchw, (0, 2, 3, 1)).astype(jnp.float32)
    xp = jnp.pad(x, ((0, 0), (1, 1), (1, 1), (0, 0)))
    x_col = jnp.concatenate(
        [xp[:, dy:dy + 32, dx:dx + 29:4, :]
         for dy in range(3) for dx in range(6)], axis=-1)
    x_col = jnp.pad(x_col, ((0, 0), (0, 0), (0, 0), (0, 10))).astype(jnp.bfloat16)

    b_blk = max(1, min(int(block_b), -(-B // 2)))
    pad_b = (-B) % b_blk
    if pad_b:
        x_col = jnp.pad(x_col, ((0, pad_b), (0, 0), (0, 0), (0, 0)))
    n_tiles = (B + pad_b) // b_blk

    const = dict(pipeline_mode=pl.Buffered(1))
    body = functools.partial(_cnn_kernel, b_blk=b_blk)
    out = pl.pallas_call(
        body,
        out_shape=jax.ShapeDtypeStruct((B + pad_b, 128), jnp.float32),
        grid=(n_tiles,),
        in_specs=[
            pl.BlockSpec((b_blk, 32, 8, 64), lambda i: (i, 0, 0, 0)),
            pl.BlockSpec((64, 256), lambda i: (0, 0), **const),
            pl.BlockSpec((1, 256), lambda i: (0, 0), **const),
            pl.BlockSpec((768, 256), lambda i: (0, 0), **const),
            pl.BlockSpec((1, 256), lambda i: (0, 0), **const),
            pl.BlockSpec((1152, 256), lambda i: (0, 0), **const),
            pl.BlockSpec((1, 256), lambda i: (0, 0), **const),
            pl.BlockSpec((4096, 512), lambda i: (0, 0), **const),
            pl.BlockSpec((1, 512), lambda i: (0, 0), **const),
            pl.BlockSpec((512, 128), lambda i: (0, 0), **const),
            pl.BlockSpec((1, 128), lambda i: (0, 0), **const),
        ],
        out_specs=pl.BlockSpec((b_blk, 128), lambda i: (i, 0)),
        scratch_shapes=[
            pltpu.VMEM((b_blk, 18, 9, 128), jnp.bfloat16),
            pltpu.VMEM((b_blk, 10, 10, 128), jnp.bfloat16),
        ],
        compiler_params=pltpu.CompilerParams(
            dimension_semantics=("parallel",),
            vmem_limit_bytes=48 * 1024 * 1024),
    )(x_col, *w)
    return out[:B, :10]


# speedup vs baseline: 2.6746x; 1.4431x over previous
import functools

import jax
import jax.numpy as jnp
from jax.experimental import pallas as pl
from jax.experimental.pallas import tpu as pltpu


def _cnn_kernel(xc_ref, w1_ref, b1_ref, w2_ref, b2_ref, w3_ref, b3_ref,
                wf1_ref, bf1_ref, wf2_ref, bf2_ref,
                o_ref,
                pad2_ref, pad3_ref, *, b_blk):
    f32 = jnp.float32
    bf16 = jnp.bfloat16
    b = b_blk

    xc = xc_ref[...].reshape(b * 32 * 8, 64)
    a1 = jnp.dot(xc, w1_ref[...], preferred_element_type=f32) + b1_ref[...]
    a1 = jnp.maximum(a1, 0.0).astype(bf16).reshape(b, 16, 2, 8, 256)
    a1 = jnp.max(a1, axis=2)
    p1e = jnp.maximum(a1[..., 0:64], a1[..., 64:128])
    p1o = jnp.maximum(a1[..., 128:192], a1[..., 192:256])

    z_row = jnp.zeros((b, 1, 9, 128), bf16)
    pad2_ref[:, 0:1] = z_row
    pad2_ref[:, 17:18] = z_row
    pad2_ref[:, 1:17, 0:1, 0:64] = jnp.zeros((b, 16, 1, 64), bf16)
    pad2_ref[:, 1:17, 8:9, 64:128] = jnp.zeros((b, 16, 1, 64), bf16)
    pad2_ref[:, 1:17, 0:8, 64:128] = p1e
    pad2_ref[:, 1:17, 1:9, 0:64] = p1o

    x2 = jnp.concatenate(
        [pad2_ref[:, dy:dy + 16, dg:dg + 8, :].reshape(b * 16 * 8, 128)
         for dy in range(3) for dg in range(2)], axis=-1)
    a2 = jnp.dot(x2, w2_ref[...], preferred_element_type=f32) + b2_ref[...]
    a2 = jnp.maximum(a2, 0.0).astype(bf16).reshape(b, 8, 2, 8, 256)
    a2 = jnp.max(a2, axis=2)
    p2 = jnp.maximum(a2[..., 0:128], a2[..., 128:256])

    z3_row = jnp.zeros((b, 1, 10, 128), bf16)
    pad3_ref[:, 0:1] = z3_row
    pad3_ref[:, 9:10] = z3_row
    z3_col = jnp.zeros((b, 8, 1, 128), bf16)
    pad3_ref[:, 1:9, 0:1, :] = z3_col
    pad3_ref[:, 1:9, 9:10, :] = z3_col
    pad3_ref[:, 1:9, 1:9, :] = p2
    x3 = jnp.concatenate(
        [pad3_ref[:, dy:dy + 8, dx:dx + 8, :].reshape(b * 64, 128)
         for dy in range(3) for dx in range(3)], axis=-1)
    a3 = jnp.dot(x3, w3_ref[...], preferred_element_type=f32) + b3_ref[...]
    a3 = jnp.maximum(a3, 0.0).astype(bf16).reshape(b * 4, 2, 4, 2, 256)
    a3 = jnp.max(a3, axis=3)
    a3 = jnp.max(a3, axis=1)
    p3 = a3.reshape(b, 4, 4, 256)

    feat = jnp.concatenate(
        [p3[:, hh, ww, :] for hh in range(4) for ww in range(4)], axis=-1)
    h1 = jnp.dot(feat, wf1_ref[...], preferred_element_type=f32) + bf1_ref[...]
    h1 = jnp.maximum(h1, 0.0).astype(bf16)

    logits = jnp.dot(h1, wf2_ref[...], preferred_element_type=f32) + bf2_ref[...]
    m = jnp.max(logits, axis=-1, keepdims=True)
    lse = m + jnp.log(jnp.sum(jnp.exp(logits - m), axis=-1, keepdims=True))
    o_ref[...] = logits - lse


def _prep_weights(conv1_w, conv1_b, conv2_w, conv2_b, conv3_w, conv3_b,
                  fc1_w, fc1_b, fc2_w, fc2_b):
    bf16, f32 = jnp.bfloat16, jnp.float32

    t1 = jnp.transpose(conv1_w, (2, 3, 1, 0))
    w1 = jnp.concatenate(
        [jnp.pad(t1, ((0, 0), (j, 3 - j), (0, 0), (0, 0))).reshape(54, 64)
         for j in range(4)], axis=1)
    w1 = jnp.pad(w1, ((0, 10), (0, 0))).astype(bf16)
    b1 = jnp.tile(conv1_b, 4).reshape(1, 256).astype(f32)

    t2 = jnp.transpose(conv2_w, (2, 3, 1, 0))
    w2 = jnp.concatenate(
        [jnp.pad(t2, ((0, 0), (j, 1 - j), (0, 0), (0, 0))).reshape(768, 128)
         for j in range(2)], axis=1).astype(bf16)
    b2 = jnp.tile(conv2_b, 2).reshape(1, 256).astype(f32)

    w3 = jnp.transpose(conv3_w, (2, 3, 1, 0)).reshape(1152, 256).astype(bf16)
    b3 = conv3_b.reshape(1, 256).astype(f32)

    wf1 = fc1_w.T.reshape(256, 4, 4, 512)
    wf1 = jnp.transpose(wf1, (1, 2, 0, 3)).reshape(4096, 512).astype(bf16)
    bf1 = fc1_b.reshape(1, 512).astype(f32)

    wf2 = jnp.pad(fc2_w.T, ((0, 0), (0, 118))).astype(bf16)
    bf2 = jnp.pad(fc2_b, (0, 118), constant_values=-1e9)
    bf2 = bf2.reshape(1, 128).astype(f32)

    return (w1, b1, w2, b2, w3, b3, wf1, bf1, wf2, bf2)


def kernel(conv1_w, conv1_b, conv2_w, conv2_b, conv3_w, conv3_b,
           fc1_w, fc1_b, fc2_w, fc2_b, x_nchw, *, block_b=16):
    w = _prep_weights(conv1_w, conv1_b, conv2_w, conv2_b, conv3_w, conv3_b,
                      fc1_w, fc1_b, fc2_w, fc2_b)
    B = x_nchw.shape[0]

    x = jnp.transpose(x_nchw, (0, 2, 3, 1)).astype(jnp.float32)
    xp = jnp.pad(x, ((0, 0), (1, 1), (1, 3), (0, 0)))
    lo = xp[:, :, 0:32, :].reshape(B, 34, 8, 4, 3)
    hi = xp[:, :, 4:36, :].reshape(B, 34, 8, 4, 3)[:, :, :, 0:2, :]
    u = jnp.concatenate([lo, hi], axis=3)
    x_col = jnp.concatenate(
        [u[:, dy:dy + 32].reshape(B, 32, 8, 18) for dy in range(3)],
        axis=-1)
    x_col = jnp.pad(x_col, ((0, 0), (0, 0), (0, 0), (0, 10))).astype(jnp.bfloat16)

    b_blk = max(1, min(int(block_b), -(-B // 2)))
    pad_b = (-B) % b_blk
    if pad_b:
        x_col = jnp.pad(x_col, ((0, pad_b), (0, 0), (0, 0), (0, 0)))
    n_tiles = (B + pad_b) // b_blk

    const = dict(pipeline_mode=pl.Buffered(1))
    body = functools.partial(_cnn_kernel, b_blk=b_blk)
    out = pl.pallas_call(
        body,
        out_shape=jax.ShapeDtypeStruct((B + pad_b, 128), jnp.float32),
        grid=(n_tiles,),
        in_specs=[
            pl.BlockSpec((b_blk, 32, 8, 64), lambda i: (i, 0, 0, 0)),
            pl.BlockSpec((64, 256), lambda i: (0, 0), **const),
            pl.BlockSpec((1, 256), lambda i: (0, 0), **const),
            pl.BlockSpec((768, 256), lambda i: (0, 0), **const),
            pl.BlockSpec((1, 256), lambda i: (0, 0), **const),
            pl.BlockSpec((1152, 256), lambda i: (0, 0), **const),
            pl.BlockSpec((1, 256), lambda i: (0, 0), **const),
            pl.BlockSpec((4096, 512), lambda i: (0, 0), **const),
            pl.BlockSpec((1, 512), lambda i: (0, 0), **const),
            pl.BlockSpec((512, 128), lambda i: (0, 0), **const),
            pl.BlockSpec((1, 128), lambda i: (0, 0), **const),
        ],
        out_specs=pl.BlockSpec((b_blk, 128), lambda i: (i, 0)),
        scratch_shapes=[
            pltpu.VMEM((b_blk, 18, 9, 128), jnp.bfloat16),
            pltpu.VMEM((b_blk, 10, 10, 128), jnp.bfloat16),
        ],
        compiler_params=pltpu.CompilerParams(
            dimension_semantics=("parallel",),
            vmem_limit_bytes=48 * 1024 * 1024),
    )(x_col, *w)
    return out[:B, :10]
